# baseline (speedup 1.0000x reference)
import jax
import jax.numpy as jnp
from jax import lax
from jax.experimental import pallas as pl
from jax.experimental.pallas import tpu as pltpu
from jax.experimental.pallas import tpu_sc as plsc

B = 4096
F = 26
V = 100000
D = 16
NPAIRS = F * (F - 1) // 2

_NC = 2
_NS = 16
_NW = _NC * _NS
_BPW = B // _NW
_RPW = _BPW * F
_CH = 128
_NCH = _RPW // _CH


def _afm_body(tab_ref, idx_ref, aux_ref, out_ref, idx_v, rows_v, ov,
              aux_v, sem):
    wid = lax.axis_index("s") * _NC + lax.axis_index("c")
    pltpu.sync_copy(idx_ref.at[wid], idx_v)
    pltpu.sync_copy(aux_ref, aux_v)
    cps = [
        pltpu.async_copy(tab_ref.at[idx_v.at[c]],
                         rows_v.at[pl.ds(c * _CH, _CH)], sem)
        for c in range(_NCH)
    ]
    for cp in cps:
        cp.wait()
    wv = aux_v[pl.ds(0, D)] * (1.0 / (2.0 * NPAIRS))
    bv = aux_v[pl.ds(D, 16)]
    lane = lax.iota(jnp.int32, 16)
    for g in range(_BPW // 16):
        def body(j, acc):
            base = (g * 16 + j) * F
            v = rows_v[base]
            s = v
            q = v * v
            for f in range(1, F):
                v = rows_v[base + f]
                s = s + v
                q = q + v * v
            x = (s * s - q) * wv
            z = jnp.sum(x)
            return jnp.where(lane == j, z, acc)

        acc = lax.fori_loop(0, 16, body, jnp.zeros(16, jnp.float32))
        ov[pl.ds(g * 16, 16)] = 1.0 / (1.0 + jnp.exp(-(acc + bv)))
    pltpu.sync_copy(ov, out_ref.at[pl.ds(wid * _BPW, _BPW)])


def kernel(dense_inputs, sparse_inputs, tables, W, b):
    del dense_inputs
    flat_idx = (sparse_inputs
                + jnp.arange(F, dtype=jnp.int32)[None, :] * V
                ).reshape(_NW, _NCH, _CH)
    tab = tables.reshape(F * V, D)
    aux = jnp.concatenate([W.reshape(D), jnp.broadcast_to(b, (16,))]
                          ).astype(jnp.float32)
    mesh = plsc.VectorSubcoreMesh(core_axis_name="c", subcore_axis_name="s")
    out = pl.kernel(
        _afm_body,
        mesh=mesh,
        compiler_params=pltpu.CompilerParams(
            needs_layout_passes=False, use_tc_tiling_on_sc=False),
        out_type=jax.ShapeDtypeStruct((B,), jnp.float32),
        scratch_types=[
            pltpu.VMEM((_NCH, _CH), jnp.int32),
            pltpu.VMEM((_RPW, D), jnp.float32),
            pltpu.VMEM((_BPW,), jnp.float32),
            pltpu.VMEM((2 * 16,), jnp.float32),
            pltpu.SemaphoreType.DMA,
        ],
    )(tab, flat_idx, aux)
    return out.reshape(B, 1)


# speedup vs baseline: 3.1839x; 3.1839x over previous
import jax
import jax.numpy as jnp
from jax import lax
from jax.experimental import pallas as pl
from jax.experimental.pallas import tpu as pltpu
from jax.experimental.pallas import tpu_sc as plsc

B = 4096
F = 26
V = 100000
D = 16
NPAIRS = F * (F - 1) // 2

_NC = 2
_NS = 16
_NW = _NC * _NS
_BPW = B // _NW
_RPW = _BPW * F


def _afm_body(tab_ref, idx_ref, aux_ref, dummy_ref, out_ref,
              idx_v, idxbuf, dplane, ov, aux_v, sem):
    wid = lax.axis_index("s") * _NC + lax.axis_index("c")
    pltpu.sync_copy(idx_ref.at[wid], idx_v)
    pltpu.sync_copy(aux_ref, aux_v)

    def gather_d(d, carry):
        db = d % 2
        off = d * V
        for f in range(F):
            for k in range(_BPW // 16):
                idxbuf[db, f, pl.ds(k * 16, 16)] = (
                    idx_v[f, pl.ds(k * 16, 16)] + off)

        @pl.when(d > 0)
        def _():
            pltpu.make_async_copy(dummy_ref, dplane.at[d - 1], sem).wait()

        for f in range(F):
            pltpu.async_copy(tab_ref.at[idxbuf.at[db, f]],
                             dplane.at[d, pl.ds(f * 128, 128)], sem)
        return carry

    lax.fori_loop(0, D, gather_d, 0)
    pltpu.make_async_copy(dummy_ref, dplane.at[D - 1], sem).wait()

    wv = aux_v[pl.ds(0, D)]
    bv = aux_v[pl.ds(D, 16)]

    def combine(k, carry):
        z = jnp.zeros(16, jnp.float32)
        for d in range(D):
            s = None
            q = None
            for f in range(F):
                v = dplane[d, pl.ds(f * 128 + k * 16, 16)]
                s = v if s is None else s + v
                q = v * v if q is None else q + v * v
            z = z + (s * s - q) * wv[d]
        z = z * (1.0 / (2.0 * NPAIRS)) + bv
        ov[pl.ds(k * 16, 16)] = 1.0 / (1.0 + jnp.exp(-z))
        return carry

    lax.fori_loop(0, _BPW // 16, combine, 0)
    pltpu.sync_copy(ov, out_ref.at[pl.ds(wid * _BPW, _BPW)])


def kernel(dense_inputs, sparse_inputs, tables, W, b):
    del dense_inputs
    tab = jnp.transpose(tables, (0, 2, 1)).reshape(-1)
    vidx = (jnp.transpose(sparse_inputs.reshape(_NW, _BPW, F), (0, 2, 1))
            + jnp.arange(F, dtype=jnp.int32)[None, :, None] * (D * V))
    aux = jnp.concatenate([W.reshape(D), jnp.broadcast_to(b, (16,))]
                          ).astype(jnp.float32)
    dummy = jnp.zeros((_RPW,), jnp.float32)
    mesh = plsc.VectorSubcoreMesh(core_axis_name="c", subcore_axis_name="s")
    out = pl.kernel(
        _afm_body,
        mesh=mesh,
        compiler_params=pltpu.CompilerParams(
            needs_layout_passes=False, use_tc_tiling_on_sc=False),
        out_type=jax.ShapeDtypeStruct((B,), jnp.float32),
        scratch_types=[
            pltpu.VMEM((F, _BPW), jnp.int32),
            pltpu.VMEM((2, F, _BPW), jnp.int32),
            pltpu.VMEM((D, _RPW), jnp.float32),
            pltpu.VMEM((_BPW,), jnp.float32),
            pltpu.VMEM((2 * 16,), jnp.float32),
            pltpu.SemaphoreType.DMA,
        ],
    )(tab, vidx, aux, dummy)
    return out.reshape(B, 1)


# speedup vs baseline: 5.3040x; 1.6659x over previous
import jax
import jax.numpy as jnp
from jax import lax
from jax.experimental import pallas as pl
from jax.experimental.pallas import tpu as pltpu
from jax.experimental.pallas import tpu_sc as plsc

B = 4096
F = 26
V = 100000
D = 16
NPAIRS = F * (F - 1) // 2

_NC = 2
_NS = 16
_NW = _NC * _NS
_BPW = B // _NW
_VR = V // _NW
_WLEN = 3328
_WCAP = 3456
_OMAX = 96640
_HCAP = 1024
_NCHK = _HCAP // 128 + 1
_ACCR = 4352


def _phase1_body(tab_ref, tail_ref, idx_ref, out_ref,
                 win, idxf, offs, rowsf, rows2d, valb, zb,
                 acc, semwin, semidx):
    cid = lax.axis_index("c")
    sid = lax.axis_index("s")
    wid = sid * _NC + cid
    lane = lax.iota(jnp.int32, 16)

    for r in range(16):
        for j in range(8):
            zb[r, pl.ds(j * 16, 16)] = jnp.zeros(16, jnp.float32)

    def zrow(r, carry):
        for j in range(6):
            valb[r, pl.ds(32 + j * 16, 16)] = jnp.zeros(16, jnp.float32)
        return carry

    lax.fori_loop(0, 128, zrow, 0)
    zbase = sid * (_ACCR // _NS)
    for k in range(_ACCR // _NS // 16):
        pltpu.sync_copy(zb, acc.at[pl.ds(zbase + k * 16, 16)])
    plsc.subcore_barrier()

    lo = wid * _VR
    hi = lo + _VR
    o = pl.multiple_of(jnp.minimum((lo >> 7) << 7, _OMAX), 128)
    has_tail = o == _OMAX

    def fire_window(f):
        pltpu.async_copy(tab_ref.at[f, :, pl.ds(o, _WLEN)],
                         win.at[:, pl.ds(0, _WLEN)], semwin)

        @pl.when(has_tail)
        def _():
            pltpu.async_copy(tail_ref.at[f], win.at[:, pl.ds(_WLEN, 128)],
                             semwin)

    def wait_window(f):
        pltpu.make_async_copy(tab_ref.at[f, :, pl.ds(o, _WLEN)],
                              win.at[:, pl.ds(0, _WLEN)], semwin).wait()

        @pl.when(has_tail)
        def _():
            pltpu.make_async_copy(tail_ref.at[f],
                                  win.at[:, pl.ds(_WLEN, 128)], semwin).wait()

    pltpu.async_copy(idx_ref.at[0], idxf.at[0], semidx)
    fire_window(0)

    def per_field(f, carry):
        db = f & 1
        pltpu.make_async_copy(idx_ref.at[f], idxf.at[db], semidx).wait()

        @pl.when(f + 1 < F)
        def _():
            pltpu.async_copy(idx_ref.at[f + 1], idxf.at[1 - db], semidx)

        def filt(kk, pos):
            vv = idxf[db, kk >> 3, pl.ds((kk & 7) * 16, 16)]
            m = (vv >= lo) & (vv < hi)
            plsc.store_compressed(offs.at[pl.ds(pos, 16)], vv - o, mask=m)
            plsc.store_compressed(rowsf.at[pl.ds(pos, 16)], kk * 16 + lane,
                                  mask=m)
            cnt = plsc.all_reduce_population_count(m)[0]
            return pos + cnt

        pos = lax.fori_loop(0, B // 16, filt, 0)
        offs[pl.ds(pos, 16)] = jnp.zeros(16, jnp.int32)
        rowsf[pl.ds(pos, 16)] = jnp.full((16,), B, jnp.int32)

        wait_window(f)

        ngrp = (pos + 15) >> 4
        nchunk = (ngrp + 7) >> 3

        def chunk(c, carry2):
            @pl.when(c < nchunk)
            def _():
                for gg in range(8):
                    g = c * 8 + gg

                    @pl.when(g < ngrp)
                    def _():
                        o16 = offs[pl.ds(g * 16, 16)]
                        r16 = rowsf[pl.ds(g * 16, 16)]
                        rows2d[c, pl.ds(gg * 16, 16)] = r16
                        for i in range(16):
                            col = plsc.load_gather(
                                win,
                                [lane, jnp.full((16,), o16[i], jnp.int32)])
                            valb[gg * 16 + i, pl.ds(0, 16)] = col
                            valb[gg * 16 + i, pl.ds(16, 16)] = col * col

                    @pl.when(g >= ngrp)
                    def _():
                        rows2d[c, pl.ds(gg * 16, 16)] = jnp.full(
                            (16,), B, jnp.int32)
                        for i in range(16):
                            valb[gg * 16 + i, pl.ds(0, 16)] = jnp.zeros(
                                16, jnp.float32)
                            valb[gg * 16 + i, pl.ds(16, 16)] = jnp.zeros(
                                16, jnp.float32)

                pltpu.sync_copy(valb, acc.at[rows2d.at[c]], add=True)
            return carry2

        lax.fori_loop(0, _NCHK, chunk, 0)

        @pl.when(f + 1 < F)
        def _():
            fire_window(f + 1)

        return carry

    lax.fori_loop(0, F, per_field, 0)
    plsc.subcore_barrier()
    ebase = sid * 256
    pltpu.sync_copy(acc.at[pl.ds(ebase, 256)],
                    out_ref.at[cid, pl.ds(ebase, 256)])


def _phase2_body(p_ref, aux_ref, out_ref, p0, p1, ov, aux_v):
    wid = lax.axis_index("s") * _NC + lax.axis_index("c")
    base = wid * _BPW
    pltpu.sync_copy(p_ref.at[0, pl.ds(base, _BPW)], p0)
    pltpu.sync_copy(p_ref.at[1, pl.ds(base, _BPW)], p1)
    pltpu.sync_copy(aux_ref, aux_v)
    wv = aux_v[pl.ds(0, D)] * (1.0 / (2.0 * NPAIRS))
    bv = aux_v[pl.ds(D, 16)]
    lane = lax.iota(jnp.int32, 16)
    for g in range(_BPW // 16):
        def body(j, acc):
            r = g * 16 + j
            s = p0[r, pl.ds(0, 16)] + p1[r, pl.ds(0, 16)]
            q = p0[r, pl.ds(16, 16)] + p1[r, pl.ds(16, 16)]
            x = (s * s - q) * wv
            z = jnp.sum(x)
            return jnp.where(lane == j, z, acc)

        acc = lax.fori_loop(0, 16, body, jnp.zeros(16, jnp.float32))
        ov[pl.ds(g * 16, 16)] = 1.0 / (1.0 + jnp.exp(-(acc + bv)))
    pltpu.sync_copy(ov, out_ref.at[pl.ds(wid * _BPW, _BPW)])


def kernel(dense_inputs, sparse_inputs, tables, W, b):
    del dense_inputs
    tab = jnp.transpose(tables, (0, 2, 1))
    tail = jnp.pad(tab[:, :, (V // 128) * 128:], ((0, 0), (0, 0), (0, 96)))
    idxT = sparse_inputs.T.reshape(F, B // 128, 128)
    aux = jnp.concatenate([W.reshape(D), jnp.broadcast_to(b, (16,))]
                          ).astype(jnp.float32)
    mesh = plsc.VectorSubcoreMesh(core_axis_name="c", subcore_axis_name="s")
    partials = pl.kernel(
        _phase1_body,
        mesh=mesh,
        compiler_params=pltpu.CompilerParams(
            needs_layout_passes=False, use_tc_tiling_on_sc=True),
        out_type=jax.ShapeDtypeStruct((_NC, B, 128), jnp.float32),
        scratch_types=[
            pltpu.VMEM((D, _WCAP), jnp.float32),
            pltpu.VMEM((2, B // 128, 128), jnp.int32),
            pltpu.VMEM((_HCAP + 16,), jnp.int32),
            pltpu.VMEM((_HCAP + 16,), jnp.int32),
            pltpu.VMEM((_NCHK, 128), jnp.int32),
            pltpu.VMEM((128, 128), jnp.float32),
            pltpu.VMEM((16, 128), jnp.float32),
            pltpu.VMEM_SHARED((_ACCR, 128), jnp.float32),
            pltpu.SemaphoreType.DMA,
            pltpu.SemaphoreType.DMA,
        ],
    )(tab, tail, idxT)
    out = pl.kernel(
        _phase2_body,
        mesh=mesh,
        compiler_params=pltpu.CompilerParams(
            needs_layout_passes=False, use_tc_tiling_on_sc=True),
        out_type=jax.ShapeDtypeStruct((B,), jnp.float32),
        scratch_types=[
            pltpu.VMEM((_BPW, 128), jnp.float32),
            pltpu.VMEM((_BPW, 128), jnp.float32),
            pltpu.VMEM((_BPW,), jnp.float32),
            pltpu.VMEM((2 * 16,), jnp.float32),
        ],
    )(partials, aux)
    return out.reshape(B, 1)
